# Initial kernel scaffold; baseline (speedup 1.0000x reference)
import jax
import jax.numpy as jnp
from jax.experimental import pallas as pl


def kernel(x, edge_index, W1, b1, W2, b2, W3, b3, W4, b4):
    raise NotImplementedError("write your pallas kernel here")


# speedup vs baseline: 16.0815x; 16.0815x over previous
import functools

import jax
import jax.numpy as jnp
from jax import lax
from jax.experimental import pallas as pl
from jax.experimental.pallas import tpu as pltpu
from jax.experimental.pallas import tpu_sc as plsc

_N = 10000
_NP = 10240
_E = 320000
_NC = 2
_NS = 16
_NW = _NC * _NS
_C = 80
_NCHUNK = 126
_EP = _NW * _NCHUNK * _C
_NCH2 = 2 * _NCHUNK
_ZCH = 128
_NZ = _NP // _NS // _ZCH


@functools.lru_cache(maxsize=None)
def _sc_mesh():
    return plsc.VectorSubcoreMesh(
        core_axis_name="c", subcore_axis_name="s", num_cores=_NC, num_subcores=_NS
    )


def _zero_fill(buf, rows, lanes):
    zero = jnp.zeros((16,), jnp.float32)

    def body(i, _):
        for k in range(lanes // 16):
            buf[i, pl.ds(k * 16, 16)] = zero
        return 0

    lax.fori_loop(0, rows, body, 0)


@functools.lru_cache(maxsize=None)
def _make_deg():
    @functools.partial(
        pl.kernel,
        out_type=jax.ShapeDtypeStruct((_NC, _NP, 16), jnp.float32),
        mesh=_sc_mesh(),
        compiler_params=pltpu.CompilerParams(use_tc_tiling_on_sc=False),
        scratch_types=[
            pltpu.VMEM((_NCHUNK, _C), jnp.int32),
            pltpu.VMEM((_C, 16), jnp.float32),
            pltpu.VMEM((_ZCH, 16), jnp.float32),
            pltpu.VMEM_SHARED((_NP, 16), jnp.float32),
        ],
    )
    def _deg_kernel(dst_hbm, deg_hbm, dst_v, ones_v, stage_v, acc_sh):
        cid = lax.axis_index("c")
        sid = lax.axis_index("s")
        wid = cid * _NS + sid

        one = jnp.ones((16,), jnp.float32)

        def fill(i, _):
            ones_v[i, :] = one
            return 0

        lax.fori_loop(0, _C, fill, 0)
        _zero_fill(stage_v, _ZCH, 16)

        base = sid * _ZCH * _NZ
        for z in range(_NZ):
            pltpu.sync_copy(stage_v, acc_sh.at[pl.ds(base + z * _ZCH, _ZCH)])
        pltpu.sync_copy(dst_hbm.at[wid], dst_v)
        plsc.subcore_barrier()

        def body(j, _):
            pltpu.sync_copy(ones_v, acc_sh.at[dst_v.at[j]], add=True)
            return 0

        lax.fori_loop(0, _NCHUNK, body, 0)
        plsc.subcore_barrier()

        for z in range(_NZ):
            pltpu.sync_copy(acc_sh.at[pl.ds(base + z * _ZCH, _ZCH)], stage_v)
            pltpu.sync_copy(stage_v, deg_hbm.at[cid, pl.ds(base + z * _ZCH, _ZCH)])

    return _deg_kernel


@functools.lru_cache(maxsize=None)
def _make_prop_split(Fh):
    @functools.partial(
        pl.kernel,
        out_type=[
            jax.ShapeDtypeStruct((_NP, Fh), jnp.float32),
            jax.ShapeDtypeStruct((_NP, Fh), jnp.float32),
        ],
        mesh=_sc_mesh(),
        compiler_params=pltpu.CompilerParams(use_tc_tiling_on_sc=False),
        scratch_types=[
            pltpu.VMEM((_NCH2, _C), jnp.int32),
            pltpu.VMEM((_NCH2, _C), jnp.int32),
            pltpu.VMEM((_C, Fh), jnp.float32),
            pltpu.VMEM((_C, Fh), jnp.float32),
            pltpu.VMEM((_ZCH, Fh), jnp.float32),
            pltpu.VMEM_SHARED((_NP, Fh), jnp.float32),
            pltpu.SemaphoreType.DMA,
            pltpu.SemaphoreType.DMA,
        ],
    )
    def prop(src_hbm, dst_hbm, g0_hbm, g1_hbm, out0_hbm, out1_hbm,
             src_v, dst_v, rows0, rows1, stage_v, acc_sh, sem0, sem1):
        cid = lax.axis_index("c")
        sid = lax.axis_index("s")

        _zero_fill(stage_v, _ZCH, Fh)
        base = sid * _ZCH * _NZ
        for z in range(_NZ):
            pltpu.sync_copy(stage_v, acc_sh.at[pl.ds(base + z * _ZCH, _ZCH)])
        pltpu.sync_copy(src_hbm.at[sid], src_v)
        pltpu.sync_copy(dst_hbm.at[sid], dst_v)
        plsc.subcore_barrier()

        def run(g_hbm, out_hbm):
            def body(jj, _):
                j0 = jj * 2
                j1 = j0 + 1
                cp0 = pltpu.async_copy(g_hbm.at[src_v.at[j0]], rows0, sem0)
                cp1 = pltpu.async_copy(g_hbm.at[src_v.at[j1]], rows1, sem1)
                cp0.wait()
                pltpu.sync_copy(rows0, acc_sh.at[dst_v.at[j0]], add=True)
                cp1.wait()
                pltpu.sync_copy(rows1, acc_sh.at[dst_v.at[j1]], add=True)
                return 0

            lax.fori_loop(0, _NCH2 // 2, body, 0)
            plsc.subcore_barrier()
            for z in range(_NZ):
                pltpu.sync_copy(acc_sh.at[pl.ds(base + z * _ZCH, _ZCH)], stage_v)
                pltpu.sync_copy(stage_v, out_hbm.at[pl.ds(base + z * _ZCH, _ZCH)])

        pl.when(cid == 0)(lambda: run(g0_hbm, out0_hbm))
        pl.when(cid == 1)(lambda: run(g1_hbm, out1_hbm))

    return prop


@functools.lru_cache(maxsize=None)
def _make_prop_full(F):
    @functools.partial(
        pl.kernel,
        out_type=jax.ShapeDtypeStruct((_NC, _NP, F), jnp.float32),
        mesh=_sc_mesh(),
        compiler_params=pltpu.CompilerParams(use_tc_tiling_on_sc=False),
        scratch_types=[
            pltpu.VMEM((_NCHUNK, _C), jnp.int32),
            pltpu.VMEM((_NCHUNK, _C), jnp.int32),
            pltpu.VMEM((_C, F), jnp.float32),
            pltpu.VMEM((_C, F), jnp.float32),
            pltpu.VMEM((_ZCH, F), jnp.float32),
            pltpu.VMEM_SHARED((_NP, F), jnp.float32),
            pltpu.SemaphoreType.DMA,
            pltpu.SemaphoreType.DMA,
        ],
    )
    def prop(src_hbm, dst_hbm, g_hbm, acc_hbm,
             src_v, dst_v, rows0, rows1, stage_v, acc_sh, sem0, sem1):
        cid = lax.axis_index("c")
        sid = lax.axis_index("s")
        wid = cid * _NS + sid

        _zero_fill(stage_v, _ZCH, F)
        base = sid * _ZCH * _NZ
        for z in range(_NZ):
            pltpu.sync_copy(stage_v, acc_sh.at[pl.ds(base + z * _ZCH, _ZCH)])
        pltpu.sync_copy(src_hbm.at[wid], src_v)
        pltpu.sync_copy(dst_hbm.at[wid], dst_v)
        plsc.subcore_barrier()

        def body(jj, _):
            j0 = jj * 2
            j1 = j0 + 1
            cp0 = pltpu.async_copy(g_hbm.at[src_v.at[j0]], rows0, sem0)
            cp1 = pltpu.async_copy(g_hbm.at[src_v.at[j1]], rows1, sem1)
            cp0.wait()
            pltpu.sync_copy(rows0, acc_sh.at[dst_v.at[j0]], add=True)
            cp1.wait()
            pltpu.sync_copy(rows1, acc_sh.at[dst_v.at[j1]], add=True)
            return 0

        lax.fori_loop(0, _NCHUNK // 2, body, 0)
        plsc.subcore_barrier()

        for z in range(_NZ):
            pltpu.sync_copy(acc_sh.at[pl.ds(base + z * _ZCH, _ZCH)], stage_v)
            pltpu.sync_copy(stage_v, acc_hbm.at[cid, pl.ds(base + z * _ZCH, _ZCH)])

    return prop


_R = 1024


def _tc_first(x, W1, deg2):
    def body(x_ref, w_ref, deg_ref, g0_ref, g1_ref, dinv_ref):
        deg = deg_ref[0] + deg_ref[1] + 1.0
        dinv = lax.rsqrt(deg)
        h = jnp.dot(x_ref[...], w_ref[...], preferred_element_type=jnp.float32)
        g = h * dinv[:, 0:1]
        g0_ref[...] = g[:, :64]
        g1_ref[...] = g[:, 64:]
        dinv_ref[...] = dinv

    return pl.pallas_call(
        body,
        grid=(_NP // _R,),
        in_specs=[
            pl.BlockSpec((_R, 128), lambda i: (i, 0)),
            pl.BlockSpec((128, 128), lambda i: (0, 0)),
            pl.BlockSpec((_NC, _R, 16), lambda i: (0, i, 0)),
        ],
        out_specs=[
            pl.BlockSpec((_R, 64), lambda i: (i, 0)),
            pl.BlockSpec((_R, 64), lambda i: (i, 0)),
            pl.BlockSpec((_R, 16), lambda i: (i, 0)),
        ],
        out_shape=[
            jax.ShapeDtypeStruct((_NP, 64), jnp.float32),
            jax.ShapeDtypeStruct((_NP, 64), jnp.float32),
            jax.ShapeDtypeStruct((_NP, 16), jnp.float32),
        ],
    )(x, W1, deg2)


def _tc_mid(acc0, acc1, g0, g1, dinv, b, W, act, F_in, F_out, split_out):
    Fh = F_in // 2
    Fo = F_out // 2 if split_out else F_out

    def body(a0_ref, a1_ref, g0_ref, g1_ref, dinv_ref, b_ref, w_ref, *o_refs):
        dv = dinv_ref[:, 0:1]
        bb = b_ref[...]
        h0 = act((a0_ref[...] + g0_ref[...]) * dv + bb[:, :Fh])
        h1 = act((a1_ref[...] + g1_ref[...]) * dv + bb[:, Fh:])
        w = w_ref[...]
        o = (jnp.dot(h0, w[:Fh, :], preferred_element_type=jnp.float32)
             + jnp.dot(h1, w[Fh:, :], preferred_element_type=jnp.float32)) * dv
        if split_out:
            o_refs[0][...] = o[:, :Fo]
            o_refs[1][...] = o[:, Fo:]
        else:
            o_refs[0][...] = o

    half = lambda: pl.BlockSpec((_R, Fh), lambda i: (i, 0))
    out_spec = pl.BlockSpec((_R, Fo), lambda i: (i, 0))
    out_shape = jax.ShapeDtypeStruct((_NP, Fo), jnp.float32)
    return pl.pallas_call(
        body,
        grid=(_NP // _R,),
        in_specs=[
            half(), half(), half(), half(),
            pl.BlockSpec((_R, 16), lambda i: (i, 0)),
            pl.BlockSpec((1, F_in), lambda i: (0, 0)),
            pl.BlockSpec((F_in, F_out), lambda i: (0, 0)),
        ],
        out_specs=[out_spec, out_spec] if split_out else [out_spec],
        out_shape=[out_shape, out_shape] if split_out else [out_shape],
    )(acc0, acc1, g0, g1, dinv, b, W)


def _tc_final(acc, g, dinv, b, F):
    def body(acc_ref, g_ref, dinv_ref, b_ref, o_ref):
        o_ref[...] = ((acc_ref[0] + acc_ref[1] + g_ref[...]) * dinv_ref[:, 0:1]
                      + b_ref[...])

    return pl.pallas_call(
        body,
        grid=(_NP // _R,),
        in_specs=[
            pl.BlockSpec((_NC, _R, F), lambda i: (0, i, 0)),
            pl.BlockSpec((_R, F), lambda i: (i, 0)),
            pl.BlockSpec((_R, 16), lambda i: (i, 0)),
            pl.BlockSpec((1, F), lambda i: (0, 0)),
        ],
        out_specs=pl.BlockSpec((_R, F), lambda i: (i, 0)),
        out_shape=jax.ShapeDtypeStruct((_NP, F), jnp.float32),
    )(acc, g, dinv, b)


def kernel(x, edge_index, W1, b1, W2, b2, W3, b3, W4, b4):
    f32 = jnp.float32
    pad_idx = jnp.full((_EP - _E,), _N, jnp.int32)
    src_p = jnp.concatenate([edge_index[0], pad_idx])
    dst_p = jnp.concatenate([edge_index[1], pad_idx])
    src32 = src_p.reshape(_NW, _NCHUNK, _C)
    dst32 = dst_p.reshape(_NW, _NCHUNK, _C)
    src16 = src_p.reshape(_NS, _NCH2, _C)
    dst16 = dst_p.reshape(_NS, _NCH2, _C)

    xp = jnp.zeros((_NP, 128), f32).at[:_N].set(x)
    W2p = jnp.zeros((128, 128), f32).at[:, :100].set(W2)
    b2p = jnp.zeros((1, 128), f32).at[0, :100].set(b2)
    W3p = jnp.zeros((128, 32), f32).at[:100, :27].set(W3)
    b3p = jnp.zeros((1, 32), f32).at[0, :27].set(b3)
    W4p = jnp.zeros((32, 16), f32).at[:27, :10].set(W4)
    b4p = jnp.zeros((1, 16), f32).at[0, :10].set(b4)
    b1r = b1.reshape(1, 128)

    deg2 = _make_deg()(dst32)
    g1a, g1b, dinv = _tc_first(xp, W1, deg2)
    a1a, a1b = _make_prop_split(64)(src16, dst16, g1a, g1b)
    g2a, g2b = _tc_mid(a1a, a1b, g1a, g1b, dinv, b1r, W2p, jnp.tanh,
                       128, 128, split_out=True)
    a2a, a2b = _make_prop_split(64)(src16, dst16, g2a, g2b)
    g3a, g3b = _tc_mid(a2a, a2b, g2a, g2b, dinv, b2p, W3p, jax.nn.relu,
                       128, 32, split_out=True)
    a3a, a3b = _make_prop_split(16)(src16, dst16, g3a, g3b)
    (g4,) = _tc_mid(a3a, a3b, g3a, g3b, dinv, b3p, W4p, jax.nn.relu,
                    32, 16, split_out=False)
    acc4 = _make_prop_full(16)(src32, dst32, g4)
    out = _tc_final(acc4, g4, dinv, b4p, 16)
    return out[:_N, :10]


# speedup vs baseline: 22.8740x; 1.4224x over previous
import functools

import jax
import jax.numpy as jnp
from jax import lax
from jax.experimental import pallas as pl
from jax.experimental.pallas import tpu as pltpu
from jax.experimental.pallas import tpu_sc as plsc

_N = 10000
_NP = 10240
_E = 320000
_NC = 2
_NS = 16
_NW = _NC * _NS
_C = 120
_NCHUNK = 84
_EP = _NW * _NCHUNK * _C
_NCH2 = 2 * _NCHUNK
_NB = 4
_ZCH = 128
_NZ = _NP // _NS // _ZCH


@functools.lru_cache(maxsize=None)
def _sc_mesh():
    return plsc.VectorSubcoreMesh(
        core_axis_name="c", subcore_axis_name="s", num_cores=_NC, num_subcores=_NS
    )


def _zero_fill(buf, rows, lanes):
    zero = jnp.zeros((16,), jnp.float32)

    def body(i, _):
        for k in range(lanes // 16):
            buf[i, pl.ds(k * 16, 16)] = zero
        return 0

    lax.fori_loop(0, rows, body, 0)


def _edge_loop(g_hbm, acc_sh, src_v, dst_v, rows, gsems, ssems, nchunk):
    for u in range(_NB):
        pltpu.async_copy(g_hbm.at[src_v.at[u]], rows[u], gsems[u])

    def body(k, _):
        base4 = k * _NB
        scats = []
        for u in range(_NB):
            jc = base4 + u
            pltpu.make_async_copy(g_hbm.at[src_v.at[jc]], rows[u], gsems[u]).wait()
            scats.append(
                pltpu.async_copy(rows[u], acc_sh.at[dst_v.at[jc]], ssems[u],
                                 add=True))
        for u in range(_NB):
            jn = base4 + _NB + u

            @pl.when(jn < nchunk)
            def _(u=u, jn=jn):
                scats[u].wait()
                pltpu.async_copy(g_hbm.at[src_v.at[jn]], rows[u], gsems[u])

        return 0

    lax.fori_loop(0, nchunk // _NB, body, 0)
    for u in range(_NB):
        pltpu.make_async_copy(
            rows[u], acc_sh.at[dst_v.at[nchunk - _NB + u]], ssems[u]).wait()


@functools.lru_cache(maxsize=None)
def _make_deg():
    @functools.partial(
        pl.kernel,
        out_type=jax.ShapeDtypeStruct((_NC, _NP, 16), jnp.float32),
        mesh=_sc_mesh(),
        compiler_params=pltpu.CompilerParams(use_tc_tiling_on_sc=False),
        scratch_types=[
            pltpu.VMEM((_NCHUNK, _C), jnp.int32),
            pltpu.VMEM((_C, 16), jnp.float32),
            pltpu.VMEM((_ZCH, 16), jnp.float32),
            pltpu.VMEM_SHARED((_NP, 16), jnp.float32),
        ],
    )
    def _deg_kernel(dst_hbm, deg_hbm, dst_v, ones_v, stage_v, acc_sh):
        cid = lax.axis_index("c")
        sid = lax.axis_index("s")
        wid = cid * _NS + sid

        one = jnp.ones((16,), jnp.float32)

        def fill(i, _):
            ones_v[i, :] = one
            return 0

        lax.fori_loop(0, _C, fill, 0)
        _zero_fill(stage_v, _ZCH, 16)

        base = sid * _ZCH * _NZ
        for z in range(_NZ):
            pltpu.sync_copy(stage_v, acc_sh.at[pl.ds(base + z * _ZCH, _ZCH)])
        pltpu.sync_copy(dst_hbm.at[wid], dst_v)
        plsc.subcore_barrier()

        def body(j, _):
            pltpu.sync_copy(ones_v, acc_sh.at[dst_v.at[j]], add=True)
            return 0

        lax.fori_loop(0, _NCHUNK, body, 0)
        plsc.subcore_barrier()

        for z in range(_NZ):
            pltpu.sync_copy(acc_sh.at[pl.ds(base + z * _ZCH, _ZCH)], stage_v)
            pltpu.sync_copy(stage_v, deg_hbm.at[cid, pl.ds(base + z * _ZCH, _ZCH)])

    return _deg_kernel


@functools.lru_cache(maxsize=None)
def _make_prop_split(Fh):
    @functools.partial(
        pl.kernel,
        out_type=[
            jax.ShapeDtypeStruct((_NP, Fh), jnp.float32),
            jax.ShapeDtypeStruct((_NP, Fh), jnp.float32),
        ],
        mesh=_sc_mesh(),
        compiler_params=pltpu.CompilerParams(use_tc_tiling_on_sc=False),
        scratch_types=[
            pltpu.VMEM((_NCH2, _C), jnp.int32),
            pltpu.VMEM((_NCH2, _C), jnp.int32),
            [pltpu.VMEM((_C, Fh), jnp.float32)] * _NB,
            pltpu.VMEM((_ZCH, Fh), jnp.float32),
            pltpu.VMEM_SHARED((_NP, Fh), jnp.float32),
            [pltpu.SemaphoreType.DMA] * _NB,
            [pltpu.SemaphoreType.DMA] * _NB,
        ],
    )
    def prop(src_hbm, dst_hbm, g0_hbm, g1_hbm, out0_hbm, out1_hbm,
             src_v, dst_v, rows, stage_v, acc_sh, gsems, ssems):
        cid = lax.axis_index("c")
        sid = lax.axis_index("s")

        _zero_fill(stage_v, _ZCH, Fh)
        base = sid * _ZCH * _NZ
        for z in range(_NZ):
            pltpu.sync_copy(stage_v, acc_sh.at[pl.ds(base + z * _ZCH, _ZCH)])
        pltpu.sync_copy(src_hbm.at[sid], src_v)
        pltpu.sync_copy(dst_hbm.at[sid], dst_v)
        plsc.subcore_barrier()

        def run(g_hbm, out_hbm):
            _edge_loop(g_hbm, acc_sh, src_v, dst_v, rows, gsems, ssems, _NCH2)
            plsc.subcore_barrier()
            for z in range(_NZ):
                pltpu.sync_copy(acc_sh.at[pl.ds(base + z * _ZCH, _ZCH)], stage_v)
                pltpu.sync_copy(stage_v, out_hbm.at[pl.ds(base + z * _ZCH, _ZCH)])

        pl.when(cid == 0)(lambda: run(g0_hbm, out0_hbm))
        pl.when(cid == 1)(lambda: run(g1_hbm, out1_hbm))

    return prop


@functools.lru_cache(maxsize=None)
def _make_prop_full(F):
    @functools.partial(
        pl.kernel,
        out_type=jax.ShapeDtypeStruct((_NC, _NP, F), jnp.float32),
        mesh=_sc_mesh(),
        compiler_params=pltpu.CompilerParams(use_tc_tiling_on_sc=False),
        scratch_types=[
            pltpu.VMEM((_NCHUNK, _C), jnp.int32),
            pltpu.VMEM((_NCHUNK, _C), jnp.int32),
            [pltpu.VMEM((_C, F), jnp.float32)] * _NB,
            pltpu.VMEM((_ZCH, F), jnp.float32),
            pltpu.VMEM_SHARED((_NP, F), jnp.float32),
            [pltpu.SemaphoreType.DMA] * _NB,
            [pltpu.SemaphoreType.DMA] * _NB,
        ],
    )
    def prop(src_hbm, dst_hbm, g_hbm, acc_hbm,
             src_v, dst_v, rows, stage_v, acc_sh, gsems, ssems):
        cid = lax.axis_index("c")
        sid = lax.axis_index("s")
        wid = cid * _NS + sid

        _zero_fill(stage_v, _ZCH, F)
        base = sid * _ZCH * _NZ
        for z in range(_NZ):
            pltpu.sync_copy(stage_v, acc_sh.at[pl.ds(base + z * _ZCH, _ZCH)])
        pltpu.sync_copy(src_hbm.at[wid], src_v)
        pltpu.sync_copy(dst_hbm.at[wid], dst_v)
        plsc.subcore_barrier()

        _edge_loop(g_hbm, acc_sh, src_v, dst_v, rows, gsems, ssems, _NCHUNK)
        plsc.subcore_barrier()

        for z in range(_NZ):
            pltpu.sync_copy(acc_sh.at[pl.ds(base + z * _ZCH, _ZCH)], stage_v)
            pltpu.sync_copy(stage_v, acc_hbm.at[cid, pl.ds(base + z * _ZCH, _ZCH)])

    return prop


_R = 1024


def _tc_first(x, W1, deg2):
    def body(x_ref, w_ref, deg_ref, g0_ref, g1_ref, dinv_ref):
        deg = deg_ref[0] + deg_ref[1] + 1.0
        dinv = lax.rsqrt(deg)
        h = jnp.dot(x_ref[...], w_ref[...], preferred_element_type=jnp.float32)
        g = h * dinv[:, 0:1]
        g0_ref[...] = g[:, :64]
        g1_ref[...] = g[:, 64:]
        dinv_ref[...] = dinv

    return pl.pallas_call(
        body,
        grid=(_NP // _R,),
        in_specs=[
            pl.BlockSpec((_R, 128), lambda i: (i, 0)),
            pl.BlockSpec((128, 128), lambda i: (0, 0)),
            pl.BlockSpec((_NC, _R, 16), lambda i: (0, i, 0)),
        ],
        out_specs=[
            pl.BlockSpec((_R, 64), lambda i: (i, 0)),
            pl.BlockSpec((_R, 64), lambda i: (i, 0)),
            pl.BlockSpec((_R, 16), lambda i: (i, 0)),
        ],
        out_shape=[
            jax.ShapeDtypeStruct((_NP, 64), jnp.float32),
            jax.ShapeDtypeStruct((_NP, 64), jnp.float32),
            jax.ShapeDtypeStruct((_NP, 16), jnp.float32),
        ],
    )(x, W1, deg2)


def _tc_mid(acc0, acc1, g0, g1, dinv, b, W, act, F_in, F_out, split_out):
    Fh = F_in // 2
    Fo = F_out // 2 if split_out else F_out

    def body(a0_ref, a1_ref, g0_ref, g1_ref, dinv_ref, b_ref, w_ref, *o_refs):
        dv = dinv_ref[:, 0:1]
        bb = b_ref[...]
        h0 = act((a0_ref[...] + g0_ref[...]) * dv + bb[:, :Fh])
        h1 = act((a1_ref[...] + g1_ref[...]) * dv + bb[:, Fh:])
        w = w_ref[...]
        o = (jnp.dot(h0, w[:Fh, :], preferred_element_type=jnp.float32)
             + jnp.dot(h1, w[Fh:, :], preferred_element_type=jnp.float32)) * dv
        if split_out:
            o_refs[0][...] = o[:, :Fo]
            o_refs[1][...] = o[:, Fo:]
        else:
            o_refs[0][...] = o

    half = lambda: pl.BlockSpec((_R, Fh), lambda i: (i, 0))
    out_spec = pl.BlockSpec((_R, Fo), lambda i: (i, 0))
    out_shape = jax.ShapeDtypeStruct((_NP, Fo), jnp.float32)
    return pl.pallas_call(
        body,
        grid=(_NP // _R,),
        in_specs=[
            half(), half(), half(), half(),
            pl.BlockSpec((_R, 16), lambda i: (i, 0)),
            pl.BlockSpec((1, F_in), lambda i: (0, 0)),
            pl.BlockSpec((F_in, F_out), lambda i: (0, 0)),
        ],
        out_specs=[out_spec, out_spec] if split_out else [out_spec],
        out_shape=[out_shape, out_shape] if split_out else [out_shape],
    )(acc0, acc1, g0, g1, dinv, b, W)


def _tc_final(acc, g, dinv, b, F):
    def body(acc_ref, g_ref, dinv_ref, b_ref, o_ref):
        o_ref[...] = ((acc_ref[0] + acc_ref[1] + g_ref[...]) * dinv_ref[:, 0:1]
                      + b_ref[...])

    return pl.pallas_call(
        body,
        grid=(_NP // _R,),
        in_specs=[
            pl.BlockSpec((_NC, _R, F), lambda i: (0, i, 0)),
            pl.BlockSpec((_R, F), lambda i: (i, 0)),
            pl.BlockSpec((_R, 16), lambda i: (i, 0)),
            pl.BlockSpec((1, F), lambda i: (0, 0)),
        ],
        out_specs=pl.BlockSpec((_R, F), lambda i: (i, 0)),
        out_shape=jax.ShapeDtypeStruct((_NP, F), jnp.float32),
    )(acc, g, dinv, b)


def kernel(x, edge_index, W1, b1, W2, b2, W3, b3, W4, b4):
    f32 = jnp.float32
    pad_idx = jnp.full((_EP - _E,), _N, jnp.int32)
    src_p = jnp.concatenate([edge_index[0], pad_idx])
    dst_p = jnp.concatenate([edge_index[1], pad_idx])
    src32 = src_p.reshape(_NW, _NCHUNK, _C)
    dst32 = dst_p.reshape(_NW, _NCHUNK, _C)
    src16 = src_p.reshape(_NS, _NCH2, _C)
    dst16 = dst_p.reshape(_NS, _NCH2, _C)

    xp = jnp.zeros((_NP, 128), f32).at[:_N].set(x)
    W2p = jnp.zeros((128, 128), f32).at[:, :100].set(W2)
    b2p = jnp.zeros((1, 128), f32).at[0, :100].set(b2)
    W3p = jnp.zeros((128, 32), f32).at[:100, :27].set(W3)
    b3p = jnp.zeros((1, 32), f32).at[0, :27].set(b3)
    W4p = jnp.zeros((32, 16), f32).at[:27, :10].set(W4)
    b4p = jnp.zeros((1, 16), f32).at[0, :10].set(b4)
    b1r = b1.reshape(1, 128)

    deg2 = _make_deg()(dst32)
    g1a, g1b, dinv = _tc_first(xp, W1, deg2)
    a1a, a1b = _make_prop_split(64)(src16, dst16, g1a, g1b)
    g2a, g2b = _tc_mid(a1a, a1b, g1a, g1b, dinv, b1r, W2p, jnp.tanh,
                       128, 128, split_out=True)
    a2a, a2b = _make_prop_split(64)(src16, dst16, g2a, g2b)
    g3a, g3b = _tc_mid(a2a, a2b, g2a, g2b, dinv, b2p, W3p, jax.nn.relu,
                       128, 32, split_out=True)
    a3a, a3b = _make_prop_split(16)(src16, dst16, g3a, g3b)
    (g4,) = _tc_mid(a3a, a3b, g3a, g3b, dinv, b3p, W4p, jax.nn.relu,
                    32, 16, split_out=False)
    acc4 = _make_prop_full(16)(src32, dst32, g4)
    out = _tc_final(acc4, g4, dinv, b4p, 16)
    return out[:_N, :10]


# speedup vs baseline: 23.4720x; 1.0261x over previous
import functools

import jax
import jax.numpy as jnp
from jax import lax
from jax.experimental import pallas as pl
from jax.experimental.pallas import tpu as pltpu
from jax.experimental.pallas import tpu_sc as plsc

_N = 10000
_NP = 10240
_E = 320000
_NC = 2
_NS = 16
_NW = _NC * _NS
_C = 120
_NCHUNK = 84
_EP = _NW * _NCHUNK * _C
_NCH2 = 2 * _NCHUNK
_NB = 6
_NBF = 4
_PASS = _NCHUNK
_ZCH = 128
_NZ = _NP // _NS // _ZCH


@functools.lru_cache(maxsize=None)
def _sc_mesh():
    return plsc.VectorSubcoreMesh(
        core_axis_name="c", subcore_axis_name="s", num_cores=_NC, num_subcores=_NS
    )


def _zero_fill(buf, rows, lanes):
    zero = jnp.zeros((16,), jnp.float32)

    def body(i, _):
        for k in range(lanes // 16):
            buf[i, pl.ds(k * 16, 16)] = zero
        return 0

    lax.fori_loop(0, rows, body, 0)


def _edge_loop(g_hbm, acc_sh, src_v, dst_v, rows, gsems, ssems, nchunk, nb):
    for u in range(nb):
        pltpu.async_copy(g_hbm.at[src_v.at[u]], rows[u], gsems[u])

    def body(k, _):
        base = k * nb
        scats = []
        for u in range(nb):
            jc = base + u
            pltpu.make_async_copy(g_hbm.at[src_v.at[jc]], rows[u], gsems[u]).wait()
            scats.append(
                pltpu.async_copy(rows[u], acc_sh.at[dst_v.at[jc]], ssems[u],
                                 add=True))
        for u in range(nb):
            jn = base + nb + u

            @pl.when(jn < nchunk)
            def _(u=u, jn=jn):
                scats[u].wait()
                pltpu.async_copy(g_hbm.at[src_v.at[jn]], rows[u], gsems[u])

        return 0

    lax.fori_loop(0, nchunk // nb, body, 0)
    for u in range(nb):
        pltpu.make_async_copy(
            rows[u], acc_sh.at[dst_v.at[nchunk - nb + u]], ssems[u]).wait()


@functools.lru_cache(maxsize=None)
def _make_deg():
    @functools.partial(
        pl.kernel,
        out_type=jax.ShapeDtypeStruct((_NC, _NP, 16), jnp.float32),
        mesh=_sc_mesh(),
        compiler_params=pltpu.CompilerParams(use_tc_tiling_on_sc=False),
        scratch_types=[
            pltpu.VMEM((_NCHUNK, _C), jnp.int32),
            pltpu.VMEM((_C, 16), jnp.float32),
            pltpu.VMEM((_ZCH, 16), jnp.float32),
            pltpu.VMEM_SHARED((_NP, 16), jnp.float32),
            [pltpu.SemaphoreType.DMA] * _NBF,
        ],
    )
    def _deg_kernel(dst_hbm, deg_hbm, dst_v, ones_v, stage_v, acc_sh, ssems):
        cid = lax.axis_index("c")
        sid = lax.axis_index("s")
        wid = cid * _NS + sid

        one = jnp.ones((16,), jnp.float32)

        def fill(i, _):
            ones_v[i, :] = one
            return 0

        lax.fori_loop(0, _C, fill, 0)
        _zero_fill(stage_v, _ZCH, 16)

        base = sid * _ZCH * _NZ
        for z in range(_NZ):
            pltpu.sync_copy(stage_v, acc_sh.at[pl.ds(base + z * _ZCH, _ZCH)])
        pltpu.sync_copy(dst_hbm.at[wid], dst_v)
        plsc.subcore_barrier()

        def body(k, _):
            base = k * _NBF
            for u in range(_NBF):
                jc = base + u

                @pl.when(jc >= _NBF)
                def _(u=u, jc=jc):
                    pltpu.make_async_copy(
                        ones_v, acc_sh.at[dst_v.at[jc - _NBF]], ssems[u]).wait()

                pltpu.async_copy(ones_v, acc_sh.at[dst_v.at[jc]], ssems[u],
                                 add=True)
            return 0

        lax.fori_loop(0, _NCHUNK // _NBF, body, 0)
        for u in range(_NBF):
            pltpu.make_async_copy(
                ones_v, acc_sh.at[dst_v.at[_NCHUNK - _NBF + u]], ssems[u]).wait()
        plsc.subcore_barrier()

        for z in range(_NZ):
            pltpu.sync_copy(acc_sh.at[pl.ds(base + z * _ZCH, _ZCH)], stage_v)
            pltpu.sync_copy(stage_v, deg_hbm.at[cid, pl.ds(base + z * _ZCH, _ZCH)])

    return _deg_kernel


@functools.lru_cache(maxsize=None)
def _make_prop_split(Fh):
    @functools.partial(
        pl.kernel,
        out_type=[
            jax.ShapeDtypeStruct((_NP, Fh), jnp.float32),
            jax.ShapeDtypeStruct((_NP, Fh), jnp.float32),
        ],
        mesh=_sc_mesh(),
        compiler_params=pltpu.CompilerParams(use_tc_tiling_on_sc=False),
        scratch_types=[
            pltpu.VMEM((_PASS, _C), jnp.int32),
            pltpu.VMEM((_PASS, _C), jnp.int32),
            [pltpu.VMEM((_C, Fh), jnp.float32)] * _NB,
            pltpu.VMEM((_ZCH, Fh), jnp.float32),
            pltpu.VMEM_SHARED((_NP, Fh), jnp.float32),
            [pltpu.SemaphoreType.DMA] * _NB,
            [pltpu.SemaphoreType.DMA] * _NB,
        ],
    )
    def prop(src_hbm, dst_hbm, g0_hbm, g1_hbm, out0_hbm, out1_hbm,
             src_v, dst_v, rows, stage_v, acc_sh, gsems, ssems):
        cid = lax.axis_index("c")
        sid = lax.axis_index("s")

        _zero_fill(stage_v, _ZCH, Fh)
        base = sid * _ZCH * _NZ
        for z in range(_NZ):
            pltpu.sync_copy(stage_v, acc_sh.at[pl.ds(base + z * _ZCH, _ZCH)])
        plsc.subcore_barrier()

        def run(g_hbm, out_hbm):
            for p in range(_NCH2 // _PASS):
                pltpu.sync_copy(src_hbm.at[sid, pl.ds(p * _PASS, _PASS)], src_v)
                pltpu.sync_copy(dst_hbm.at[sid, pl.ds(p * _PASS, _PASS)], dst_v)
                _edge_loop(g_hbm, acc_sh, src_v, dst_v, rows, gsems, ssems,
                           _PASS, _NB)
            plsc.subcore_barrier()
            for z in range(_NZ):
                pltpu.sync_copy(acc_sh.at[pl.ds(base + z * _ZCH, _ZCH)], stage_v)
                pltpu.sync_copy(stage_v, out_hbm.at[pl.ds(base + z * _ZCH, _ZCH)])

        pl.when(cid == 0)(lambda: run(g0_hbm, out0_hbm))
        pl.when(cid == 1)(lambda: run(g1_hbm, out1_hbm))

    return prop


@functools.lru_cache(maxsize=None)
def _make_prop_full(F):
    @functools.partial(
        pl.kernel,
        out_type=jax.ShapeDtypeStruct((_NC, _NP, F), jnp.float32),
        mesh=_sc_mesh(),
        compiler_params=pltpu.CompilerParams(use_tc_tiling_on_sc=False),
        scratch_types=[
            pltpu.VMEM((_NCHUNK, _C), jnp.int32),
            pltpu.VMEM((_NCHUNK, _C), jnp.int32),
            [pltpu.VMEM((_C, F), jnp.float32)] * _NBF,
            pltpu.VMEM((_ZCH, F), jnp.float32),
            pltpu.VMEM_SHARED((_NP, F), jnp.float32),
            [pltpu.SemaphoreType.DMA] * _NBF,
            [pltpu.SemaphoreType.DMA] * _NBF,
        ],
    )
    def prop(src_hbm, dst_hbm, g_hbm, acc_hbm,
             src_v, dst_v, rows, stage_v, acc_sh, gsems, ssems):
        cid = lax.axis_index("c")
        sid = lax.axis_index("s")
        wid = cid * _NS + sid

        _zero_fill(stage_v, _ZCH, F)
        base = sid * _ZCH * _NZ
        for z in range(_NZ):
            pltpu.sync_copy(stage_v, acc_sh.at[pl.ds(base + z * _ZCH, _ZCH)])
        pltpu.sync_copy(src_hbm.at[wid], src_v)
        pltpu.sync_copy(dst_hbm.at[wid], dst_v)
        plsc.subcore_barrier()

        _edge_loop(g_hbm, acc_sh, src_v, dst_v, rows, gsems, ssems, _NCHUNK,
                   _NBF)
        plsc.subcore_barrier()

        for z in range(_NZ):
            pltpu.sync_copy(acc_sh.at[pl.ds(base + z * _ZCH, _ZCH)], stage_v)
            pltpu.sync_copy(stage_v, acc_hbm.at[cid, pl.ds(base + z * _ZCH, _ZCH)])

    return prop


_R = 1024


def _tc_first(x, W1, deg2):
    def body(x_ref, w_ref, deg_ref, g0_ref, g1_ref, dinv_ref):
        deg = deg_ref[0] + deg_ref[1] + 1.0
        dinv = lax.rsqrt(deg)
        h = jnp.dot(x_ref[...], w_ref[...], preferred_element_type=jnp.float32)
        g = h * dinv[:, 0:1]
        g0_ref[...] = g[:, :64]
        g1_ref[...] = g[:, 64:]
        dinv_ref[...] = dinv

    return pl.pallas_call(
        body,
        grid=(_NP // _R,),
        in_specs=[
            pl.BlockSpec((_R, 128), lambda i: (i, 0)),
            pl.BlockSpec((128, 128), lambda i: (0, 0)),
            pl.BlockSpec((_NC, _R, 16), lambda i: (0, i, 0)),
        ],
        out_specs=[
            pl.BlockSpec((_R, 64), lambda i: (i, 0)),
            pl.BlockSpec((_R, 64), lambda i: (i, 0)),
            pl.BlockSpec((_R, 16), lambda i: (i, 0)),
        ],
        out_shape=[
            jax.ShapeDtypeStruct((_NP, 64), jnp.float32),
            jax.ShapeDtypeStruct((_NP, 64), jnp.float32),
            jax.ShapeDtypeStruct((_NP, 16), jnp.float32),
        ],
    )(x, W1, deg2)


def _tc_mid(acc0, acc1, g0, g1, dinv, b, W, act, F_in, F_out, split_out):
    Fh = F_in // 2
    Fo = F_out // 2 if split_out else F_out

    def body(a0_ref, a1_ref, g0_ref, g1_ref, dinv_ref, b_ref, w_ref, *o_refs):
        dv = dinv_ref[:, 0:1]
        bb = b_ref[...]
        h0 = act((a0_ref[...] + g0_ref[...]) * dv + bb[:, :Fh])
        h1 = act((a1_ref[...] + g1_ref[...]) * dv + bb[:, Fh:])
        w = w_ref[...]
        o = (jnp.dot(h0, w[:Fh, :], preferred_element_type=jnp.float32)
             + jnp.dot(h1, w[Fh:, :], preferred_element_type=jnp.float32)) * dv
        if split_out:
            o_refs[0][...] = o[:, :Fo]
            o_refs[1][...] = o[:, Fo:]
        else:
            o_refs[0][...] = o

    half = lambda: pl.BlockSpec((_R, Fh), lambda i: (i, 0))
    out_spec = pl.BlockSpec((_R, Fo), lambda i: (i, 0))
    out_shape = jax.ShapeDtypeStruct((_NP, Fo), jnp.float32)
    return pl.pallas_call(
        body,
        grid=(_NP // _R,),
        in_specs=[
            half(), half(), half(), half(),
            pl.BlockSpec((_R, 16), lambda i: (i, 0)),
            pl.BlockSpec((1, F_in), lambda i: (0, 0)),
            pl.BlockSpec((F_in, F_out), lambda i: (0, 0)),
        ],
        out_specs=[out_spec, out_spec] if split_out else [out_spec],
        out_shape=[out_shape, out_shape] if split_out else [out_shape],
    )(acc0, acc1, g0, g1, dinv, b, W)


def _tc_final(acc, g, dinv, b, F):
    def body(acc_ref, g_ref, dinv_ref, b_ref, o_ref):
        o_ref[...] = ((acc_ref[0] + acc_ref[1] + g_ref[...]) * dinv_ref[:, 0:1]
                      + b_ref[...])

    return pl.pallas_call(
        body,
        grid=(_NP // _R,),
        in_specs=[
            pl.BlockSpec((_NC, _R, F), lambda i: (0, i, 0)),
            pl.BlockSpec((_R, F), lambda i: (i, 0)),
            pl.BlockSpec((_R, 16), lambda i: (i, 0)),
            pl.BlockSpec((1, F), lambda i: (0, 0)),
        ],
        out_specs=pl.BlockSpec((_R, F), lambda i: (i, 0)),
        out_shape=jax.ShapeDtypeStruct((_NP, F), jnp.float32),
    )(acc, g, dinv, b)


def kernel(x, edge_index, W1, b1, W2, b2, W3, b3, W4, b4):
    f32 = jnp.float32
    pad_idx = jnp.full((_EP - _E,), _N, jnp.int32)
    src_p = jnp.concatenate([edge_index[0], pad_idx])
    dst_p = jnp.concatenate([edge_index[1], pad_idx])
    src32 = src_p.reshape(_NW, _NCHUNK, _C)
    dst32 = dst_p.reshape(_NW, _NCHUNK, _C)
    src16 = src_p.reshape(_NS, _NCH2, _C)
    dst16 = dst_p.reshape(_NS, _NCH2, _C)

    xp = jnp.zeros((_NP, 128), f32).at[:_N].set(x)
    W2p = jnp.zeros((128, 128), f32).at[:, :100].set(W2)
    b2p = jnp.zeros((1, 128), f32).at[0, :100].set(b2)
    W3p = jnp.zeros((128, 32), f32).at[:100, :27].set(W3)
    b3p = jnp.zeros((1, 32), f32).at[0, :27].set(b3)
    W4p = jnp.zeros((32, 16), f32).at[:27, :10].set(W4)
    b4p = jnp.zeros((1, 16), f32).at[0, :10].set(b4)
    b1r = b1.reshape(1, 128)

    deg2 = _make_deg()(dst32)
    g1a, g1b, dinv = _tc_first(xp, W1, deg2)
    a1a, a1b = _make_prop_split(64)(src16, dst16, g1a, g1b)
    g2a, g2b = _tc_mid(a1a, a1b, g1a, g1b, dinv, b1r, W2p, jnp.tanh,
                       128, 128, split_out=True)
    a2a, a2b = _make_prop_split(64)(src16, dst16, g2a, g2b)
    g3a, g3b = _tc_mid(a2a, a2b, g2a, g2b, dinv, b2p, W3p, jax.nn.relu,
                       128, 32, split_out=True)
    a3a, a3b = _make_prop_split(16)(src16, dst16, g3a, g3b)
    (g4,) = _tc_mid(a3a, a3b, g3a, g3b, dinv, b3p, W4p, jax.nn.relu,
                    32, 16, split_out=False)
    acc4 = _make_prop_full(16)(src32, dst32, g4)
    out = _tc_final(acc4, g4, dinv, b4p, 16)
    return out[:_N, :10]


# speedup vs baseline: 24.2469x; 1.0330x over previous
import functools

import jax
import jax.numpy as jnp
from jax import lax
from jax.experimental import pallas as pl
from jax.experimental.pallas import tpu as pltpu
from jax.experimental.pallas import tpu_sc as plsc

_N = 10000
_NP = 10240
_E = 320000
_NC = 2
_NS = 16
_NW = _NC * _NS
_C = 120
_NCHUNK = 84
_EP = _NW * _NCHUNK * _C
_NCH2 = 2 * _NCHUNK
_NB = 6
_NBF = 4
_PASS = _NCHUNK
_ZCH = 128
_NZ = _NP // _NS // _ZCH


@functools.lru_cache(maxsize=None)
def _sc_mesh():
    return plsc.VectorSubcoreMesh(
        core_axis_name="c", subcore_axis_name="s", num_cores=_NC, num_subcores=_NS
    )


def _zero_fill(buf, rows, lanes):
    zero = jnp.zeros((16,), jnp.float32)

    def body(i, _):
        for k in range(lanes // 16):
            buf[i, pl.ds(k * 16, 16)] = zero
        return 0

    lax.fori_loop(0, rows, body, 0)


def _edge_loop(g_hbm, acc_sh, src_v, dst_v, rows, gsems, ssems, nchunk, nb):
    for u in range(nb):
        pltpu.async_copy(g_hbm.at[src_v.at[u]], rows[u], gsems[u])

    def body(k, _):
        base = k * nb
        scats = []
        for u in range(nb):
            jc = base + u
            pltpu.make_async_copy(g_hbm.at[src_v.at[jc]], rows[u], gsems[u]).wait()
            scats.append(
                pltpu.async_copy(rows[u], acc_sh.at[dst_v.at[jc]], ssems[u],
                                 add=True))
        for u in range(nb):
            jn = base + nb + u

            @pl.when(jn < nchunk)
            def _(u=u, jn=jn):
                scats[u].wait()
                pltpu.async_copy(g_hbm.at[src_v.at[jn]], rows[u], gsems[u])

        return 0

    lax.fori_loop(0, nchunk // nb, body, 0)
    for u in range(nb):
        pltpu.make_async_copy(
            rows[u], acc_sh.at[dst_v.at[nchunk - nb + u]], ssems[u]).wait()


@functools.lru_cache(maxsize=None)
def _make_deg():
    @functools.partial(
        pl.kernel,
        out_type=jax.ShapeDtypeStruct((_NC, _NP, 16), jnp.float32),
        mesh=_sc_mesh(),
        compiler_params=pltpu.CompilerParams(use_tc_tiling_on_sc=False),
        scratch_types=[
            pltpu.VMEM((_NCHUNK, _C), jnp.int32),
            pltpu.VMEM((_C, 16), jnp.float32),
            pltpu.VMEM((_ZCH, 16), jnp.float32),
            pltpu.VMEM_SHARED((_NP, 16), jnp.float32),
            [pltpu.SemaphoreType.DMA] * _NBF,
        ],
    )
    def _deg_kernel(dst_hbm, deg_hbm, dst_v, ones_v, stage_v, acc_sh, ssems):
        cid = lax.axis_index("c")
        sid = lax.axis_index("s")
        wid = cid * _NS + sid

        one = jnp.ones((16,), jnp.float32)

        def fill(i, _):
            ones_v[i, :] = one
            return 0

        lax.fori_loop(0, _C, fill, 0)
        _zero_fill(stage_v, _ZCH, 16)

        base = sid * _ZCH * _NZ
        for z in range(_NZ):
            pltpu.sync_copy(stage_v, acc_sh.at[pl.ds(base + z * _ZCH, _ZCH)])
        pltpu.sync_copy(dst_hbm.at[wid], dst_v)
        plsc.subcore_barrier()

        def body(k, _):
            base = k * _NBF
            for u in range(_NBF):
                jc = base + u

                @pl.when(jc >= _NBF)
                def _(u=u, jc=jc):
                    pltpu.make_async_copy(
                        ones_v, acc_sh.at[dst_v.at[jc - _NBF]], ssems[u]).wait()

                pltpu.async_copy(ones_v, acc_sh.at[dst_v.at[jc]], ssems[u],
                                 add=True)
            return 0

        lax.fori_loop(0, _NCHUNK // _NBF, body, 0)
        for u in range(_NBF):
            pltpu.make_async_copy(
                ones_v, acc_sh.at[dst_v.at[_NCHUNK - _NBF + u]], ssems[u]).wait()
        plsc.subcore_barrier()

        for z in range(_NZ):
            pltpu.sync_copy(acc_sh.at[pl.ds(base + z * _ZCH, _ZCH)], stage_v)
            pltpu.sync_copy(stage_v, deg_hbm.at[cid, pl.ds(base + z * _ZCH, _ZCH)])

    return _deg_kernel


@functools.lru_cache(maxsize=None)
def _make_prop_split(Fh, nb=_NB):
    @functools.partial(
        pl.kernel,
        out_type=[
            jax.ShapeDtypeStruct((_NP, Fh), jnp.float32),
            jax.ShapeDtypeStruct((_NP, Fh), jnp.float32),
        ],
        mesh=_sc_mesh(),
        compiler_params=pltpu.CompilerParams(use_tc_tiling_on_sc=False),
        scratch_types=[
            pltpu.VMEM((_PASS, _C), jnp.int32),
            pltpu.VMEM((_PASS, _C), jnp.int32),
            [pltpu.VMEM((_C, Fh), jnp.float32)] * nb,
            pltpu.VMEM((_ZCH, Fh), jnp.float32),
            pltpu.VMEM_SHARED((_NP, Fh), jnp.float32),
            [pltpu.SemaphoreType.DMA] * nb,
            [pltpu.SemaphoreType.DMA] * nb,
        ],
    )
    def prop(src_hbm, dst_hbm, g0_hbm, g1_hbm, out0_hbm, out1_hbm,
             src_v, dst_v, rows, stage_v, acc_sh, gsems, ssems):
        cid = lax.axis_index("c")
        sid = lax.axis_index("s")

        _zero_fill(stage_v, _ZCH, Fh)
        base = sid * _ZCH * _NZ
        for z in range(_NZ):
            pltpu.sync_copy(stage_v, acc_sh.at[pl.ds(base + z * _ZCH, _ZCH)])
        plsc.subcore_barrier()

        def run(g_hbm, out_hbm):
            for p in range(_NCH2 // _PASS):
                pltpu.sync_copy(src_hbm.at[sid, pl.ds(p * _PASS, _PASS)], src_v)
                pltpu.sync_copy(dst_hbm.at[sid, pl.ds(p * _PASS, _PASS)], dst_v)
                _edge_loop(g_hbm, acc_sh, src_v, dst_v, rows, gsems, ssems,
                           _PASS, nb)
            plsc.subcore_barrier()
            for z in range(_NZ):
                pltpu.sync_copy(acc_sh.at[pl.ds(base + z * _ZCH, _ZCH)], stage_v)
                pltpu.sync_copy(stage_v, out_hbm.at[pl.ds(base + z * _ZCH, _ZCH)])

        pl.when(cid == 0)(lambda: run(g0_hbm, out0_hbm))
        pl.when(cid == 1)(lambda: run(g1_hbm, out1_hbm))

    return prop


@functools.lru_cache(maxsize=None)
def _make_prop_full(F, nb=_NBF):
    @functools.partial(
        pl.kernel,
        out_type=jax.ShapeDtypeStruct((_NC, _NP, F), jnp.float32),
        mesh=_sc_mesh(),
        compiler_params=pltpu.CompilerParams(use_tc_tiling_on_sc=False),
        scratch_types=[
            pltpu.VMEM((_NCHUNK, _C), jnp.int32),
            pltpu.VMEM((_NCHUNK, _C), jnp.int32),
            [pltpu.VMEM((_C, F), jnp.float32)] * nb,
            pltpu.VMEM((_ZCH, F), jnp.float32),
            pltpu.VMEM_SHARED((_NP, F), jnp.float32),
            [pltpu.SemaphoreType.DMA] * nb,
            [pltpu.SemaphoreType.DMA] * nb,
        ],
    )
    def prop(src_hbm, dst_hbm, g_hbm, acc_hbm,
             src_v, dst_v, rows, stage_v, acc_sh, gsems, ssems):
        cid = lax.axis_index("c")
        sid = lax.axis_index("s")
        wid = cid * _NS + sid

        _zero_fill(stage_v, _ZCH, F)
        base = sid * _ZCH * _NZ
        for z in range(_NZ):
            pltpu.sync_copy(stage_v, acc_sh.at[pl.ds(base + z * _ZCH, _ZCH)])
        pltpu.sync_copy(src_hbm.at[wid], src_v)
        pltpu.sync_copy(dst_hbm.at[wid], dst_v)
        plsc.subcore_barrier()

        _edge_loop(g_hbm, acc_sh, src_v, dst_v, rows, gsems, ssems, _NCHUNK,
                   nb)
        plsc.subcore_barrier()

        for z in range(_NZ):
            pltpu.sync_copy(acc_sh.at[pl.ds(base + z * _ZCH, _ZCH)], stage_v)
            pltpu.sync_copy(stage_v, acc_hbm.at[cid, pl.ds(base + z * _ZCH, _ZCH)])

    return prop


_R = 2048


def _tc_first(x, W1, deg2):
    def body(x_ref, w_ref, deg_ref, g0_ref, g1_ref, dinv_ref):
        deg = deg_ref[0] + deg_ref[1] + 1.0
        dinv = lax.rsqrt(deg)
        h = jnp.dot(x_ref[...], w_ref[...], preferred_element_type=jnp.float32)
        g = h * dinv[:, 0:1]
        g0_ref[...] = g[:, :64]
        g1_ref[...] = g[:, 64:]
        dinv_ref[...] = dinv

    return pl.pallas_call(
        body,
        grid=(_NP // _R,),
        in_specs=[
            pl.BlockSpec((_R, 128), lambda i: (i, 0)),
            pl.BlockSpec((128, 128), lambda i: (0, 0)),
            pl.BlockSpec((_NC, _R, 16), lambda i: (0, i, 0)),
        ],
        out_specs=[
            pl.BlockSpec((_R, 64), lambda i: (i, 0)),
            pl.BlockSpec((_R, 64), lambda i: (i, 0)),
            pl.BlockSpec((_R, 16), lambda i: (i, 0)),
        ],
        out_shape=[
            jax.ShapeDtypeStruct((_NP, 64), jnp.float32),
            jax.ShapeDtypeStruct((_NP, 64), jnp.float32),
            jax.ShapeDtypeStruct((_NP, 16), jnp.float32),
        ],
    )(x, W1, deg2)


def _tc_mid(acc0, acc1, g0, g1, dinv, b, W, act, F_in, F_out, split_out):
    Fh = F_in // 2
    Fo = F_out // 2 if split_out else F_out

    def body(a0_ref, a1_ref, g0_ref, g1_ref, dinv_ref, b_ref, w_ref, *o_refs):
        dv = dinv_ref[:, 0:1]
        bb = b_ref[...]
        h0 = act((a0_ref[...] + g0_ref[...]) * dv + bb[:, :Fh])
        h1 = act((a1_ref[...] + g1_ref[...]) * dv + bb[:, Fh:])
        w = w_ref[...]
        o = (jnp.dot(h0, w[:Fh, :], preferred_element_type=jnp.float32)
             + jnp.dot(h1, w[Fh:, :], preferred_element_type=jnp.float32)) * dv
        if split_out:
            o_refs[0][...] = o[:, :Fo]
            o_refs[1][...] = o[:, Fo:]
        else:
            o_refs[0][...] = o

    half = lambda: pl.BlockSpec((_R, Fh), lambda i: (i, 0))
    out_spec = pl.BlockSpec((_R, Fo), lambda i: (i, 0))
    out_shape = jax.ShapeDtypeStruct((_NP, Fo), jnp.float32)
    return pl.pallas_call(
        body,
        grid=(_NP // _R,),
        in_specs=[
            half(), half(), half(), half(),
            pl.BlockSpec((_R, 16), lambda i: (i, 0)),
            pl.BlockSpec((1, F_in), lambda i: (0, 0)),
            pl.BlockSpec((F_in, F_out), lambda i: (0, 0)),
        ],
        out_specs=[out_spec, out_spec] if split_out else [out_spec],
        out_shape=[out_shape, out_shape] if split_out else [out_shape],
    )(acc0, acc1, g0, g1, dinv, b, W)


def _tc_final(acc, g, dinv, b, F):
    def body(acc_ref, g_ref, dinv_ref, b_ref, o_ref):
        o_ref[...] = ((acc_ref[0] + acc_ref[1] + g_ref[...]) * dinv_ref[:, 0:1]
                      + b_ref[...])

    return pl.pallas_call(
        body,
        grid=(_NP // _R,),
        in_specs=[
            pl.BlockSpec((_NC, _R, F), lambda i: (0, i, 0)),
            pl.BlockSpec((_R, F), lambda i: (i, 0)),
            pl.BlockSpec((_R, 16), lambda i: (i, 0)),
            pl.BlockSpec((1, F), lambda i: (0, 0)),
        ],
        out_specs=pl.BlockSpec((_R, F), lambda i: (i, 0)),
        out_shape=jax.ShapeDtypeStruct((_NP, F), jnp.float32),
    )(acc, g, dinv, b)


def kernel(x, edge_index, W1, b1, W2, b2, W3, b3, W4, b4):
    f32 = jnp.float32
    pad_idx = jnp.full((_EP - _E,), _N, jnp.int32)
    src_p = jnp.concatenate([edge_index[0], pad_idx])
    dst_p = jnp.concatenate([edge_index[1], pad_idx])
    src32 = src_p.reshape(_NW, _NCHUNK, _C)
    dst32 = dst_p.reshape(_NW, _NCHUNK, _C)
    src16 = src_p.reshape(_NS, _NCH2, _C)
    dst16 = dst_p.reshape(_NS, _NCH2, _C)

    xp = jnp.zeros((_NP, 128), f32).at[:_N].set(x)
    W2p = jnp.zeros((128, 128), f32).at[:, :100].set(W2)
    b2p = jnp.zeros((1, 128), f32).at[0, :100].set(b2)
    W3p = jnp.zeros((128, 32), f32).at[:100, :27].set(W3)
    b3p = jnp.zeros((1, 32), f32).at[0, :27].set(b3)
    W4p = jnp.zeros((32, 16), f32).at[:27, :10].set(W4)
    b4p = jnp.zeros((1, 16), f32).at[0, :10].set(b4)
    b1r = b1.reshape(1, 128)

    deg2 = _make_deg()(dst32)
    g1a, g1b, dinv = _tc_first(xp, W1, deg2)
    a1a, a1b = _make_prop_split(64)(src16, dst16, g1a, g1b)
    g2a, g2b = _tc_mid(a1a, a1b, g1a, g1b, dinv, b1r, W2p, jnp.tanh,
                       128, 128, split_out=True)
    a2a, a2b = _make_prop_split(64)(src16, dst16, g2a, g2b)
    g3a, g3b = _tc_mid(a2a, a2b, g2a, g2b, dinv, b2p, W3p, jax.nn.relu,
                       128, 32, split_out=True)
    a3a, a3b = _make_prop_split(16, nb=12)(src16, dst16, g3a, g3b)
    (g4,) = _tc_mid(a3a, a3b, g3a, g3b, dinv, b3p, W4p, jax.nn.relu,
                    32, 16, split_out=False)
    acc4 = _make_prop_full(16, nb=12)(src32, dst32, g4)
    out = _tc_final(acc4, g4, dinv, b4p, 16)
    return out[:_N, :10]


# speedup vs baseline: 24.2517x; 1.0002x over previous
import functools

import jax
import jax.numpy as jnp
from jax import lax
from jax.experimental import pallas as pl
from jax.experimental.pallas import tpu as pltpu
from jax.experimental.pallas import tpu_sc as plsc

_N = 10000
_NP = 10240
_E = 320000
_NC = 2
_NS = 16
_NW = _NC * _NS
_C = 120
_NCHUNK = 84
_EP = _NW * _NCHUNK * _C
_NCH2 = 2 * _NCHUNK
_NB = 6
_NBF = 4
_PASS = _NCHUNK
_ZCH = 128
_NZ = _NP // _NS // _ZCH


@functools.lru_cache(maxsize=None)
def _sc_mesh():
    return plsc.VectorSubcoreMesh(
        core_axis_name="c", subcore_axis_name="s", num_cores=_NC, num_subcores=_NS
    )


def _zero_fill(buf, rows, lanes):
    zero = jnp.zeros((16,), jnp.float32)

    def body(i, _):
        for k in range(lanes // 16):
            buf[i, pl.ds(k * 16, 16)] = zero
        return 0

    lax.fori_loop(0, rows, body, 0)


def _edge_loop(g_hbm, acc_sh, src_v, dst_v, rows, gsems, ssems, nchunk, nb):
    for u in range(nb):
        pltpu.async_copy(g_hbm.at[src_v.at[u]], rows[u], gsems[u])

    def body(k, _):
        base = k * nb
        scats = []
        for u in range(nb):
            jc = base + u
            pltpu.make_async_copy(g_hbm.at[src_v.at[jc]], rows[u], gsems[u]).wait()
            scats.append(
                pltpu.async_copy(rows[u], acc_sh.at[dst_v.at[jc]], ssems[u],
                                 add=True))
        for u in range(nb):
            jn = base + nb + u

            @pl.when(jn < nchunk)
            def _(u=u, jn=jn):
                scats[u].wait()
                pltpu.async_copy(g_hbm.at[src_v.at[jn]], rows[u], gsems[u])

        return 0

    lax.fori_loop(0, nchunk // nb, body, 0)
    for u in range(nb):
        pltpu.make_async_copy(
            rows[u], acc_sh.at[dst_v.at[nchunk - nb + u]], ssems[u]).wait()


@functools.lru_cache(maxsize=None)
def _make_deg():
    @functools.partial(
        pl.kernel,
        out_type=jax.ShapeDtypeStruct((_NC, _NP, 16), jnp.float32),
        mesh=_sc_mesh(),
        compiler_params=pltpu.CompilerParams(use_tc_tiling_on_sc=False),
        scratch_types=[
            pltpu.VMEM((_NCHUNK, _C), jnp.int32),
            pltpu.VMEM((_C, 16), jnp.float32),
            pltpu.VMEM((_ZCH, 16), jnp.float32),
            pltpu.VMEM_SHARED((_NP, 16), jnp.float32),
            [pltpu.SemaphoreType.DMA] * _NBF,
        ],
    )
    def _deg_kernel(dst_hbm, deg_hbm, dst_v, ones_v, stage_v, acc_sh, ssems):
        cid = lax.axis_index("c")
        sid = lax.axis_index("s")
        wid = cid * _NS + sid

        one = jnp.ones((16,), jnp.float32)

        def fill(i, _):
            ones_v[i, :] = one
            return 0

        lax.fori_loop(0, _C, fill, 0)
        _zero_fill(stage_v, _ZCH, 16)

        base = sid * _ZCH * _NZ
        for z in range(_NZ):
            pltpu.sync_copy(stage_v, acc_sh.at[pl.ds(base + z * _ZCH, _ZCH)])
        pltpu.sync_copy(dst_hbm.at[wid], dst_v)
        plsc.subcore_barrier()

        def body(k, _):
            base = k * _NBF
            for u in range(_NBF):
                jc = base + u

                @pl.when(jc >= _NBF)
                def _(u=u, jc=jc):
                    pltpu.make_async_copy(
                        ones_v, acc_sh.at[dst_v.at[jc - _NBF]], ssems[u]).wait()

                pltpu.async_copy(ones_v, acc_sh.at[dst_v.at[jc]], ssems[u],
                                 add=True)
            return 0

        lax.fori_loop(0, _NCHUNK // _NBF, body, 0)
        for u in range(_NBF):
            pltpu.make_async_copy(
                ones_v, acc_sh.at[dst_v.at[_NCHUNK - _NBF + u]], ssems[u]).wait()
        plsc.subcore_barrier()

        for z in range(_NZ):
            pltpu.sync_copy(acc_sh.at[pl.ds(base + z * _ZCH, _ZCH)], stage_v)
            pltpu.sync_copy(stage_v, deg_hbm.at[cid, pl.ds(base + z * _ZCH, _ZCH)])

    return _deg_kernel


@functools.lru_cache(maxsize=None)
def _make_prop_split(Fh, nb=_NB):
    @functools.partial(
        pl.kernel,
        out_type=[
            jax.ShapeDtypeStruct((_NP, Fh), jnp.float32),
            jax.ShapeDtypeStruct((_NP, Fh), jnp.float32),
        ],
        mesh=_sc_mesh(),
        compiler_params=pltpu.CompilerParams(use_tc_tiling_on_sc=False),
        scratch_types=[
            pltpu.VMEM((_PASS, _C), jnp.int32),
            pltpu.VMEM((_PASS, _C), jnp.int32),
            [pltpu.VMEM((_C, Fh), jnp.float32)] * nb,
            pltpu.VMEM((_ZCH, Fh), jnp.float32),
            pltpu.VMEM_SHARED((_NP, Fh), jnp.float32),
            [pltpu.SemaphoreType.DMA] * nb,
            [pltpu.SemaphoreType.DMA] * nb,
        ],
    )
    def prop(src_hbm, dst_hbm, g0_hbm, g1_hbm, out0_hbm, out1_hbm,
             src_v, dst_v, rows, stage_v, acc_sh, gsems, ssems):
        cid = lax.axis_index("c")
        sid = lax.axis_index("s")

        _zero_fill(stage_v, _ZCH, Fh)
        base = sid * _ZCH * _NZ
        for z in range(_NZ):
            pltpu.sync_copy(stage_v, acc_sh.at[pl.ds(base + z * _ZCH, _ZCH)])
        plsc.subcore_barrier()

        def run(g_hbm, out_hbm):
            for p in range(_NCH2 // _PASS):
                pltpu.sync_copy(src_hbm.at[sid, pl.ds(p * _PASS, _PASS)], src_v)
                pltpu.sync_copy(dst_hbm.at[sid, pl.ds(p * _PASS, _PASS)], dst_v)
                _edge_loop(g_hbm, acc_sh, src_v, dst_v, rows, gsems, ssems,
                           _PASS, nb)
            plsc.subcore_barrier()
            for z in range(_NZ):
                pltpu.sync_copy(acc_sh.at[pl.ds(base + z * _ZCH, _ZCH)], stage_v)
                pltpu.sync_copy(stage_v, out_hbm.at[pl.ds(base + z * _ZCH, _ZCH)])

        pl.when(cid == 0)(lambda: run(g0_hbm, out0_hbm))
        pl.when(cid == 1)(lambda: run(g1_hbm, out1_hbm))

    return prop


@functools.lru_cache(maxsize=None)
def _make_prop_full(F, nb=_NBF):
    @functools.partial(
        pl.kernel,
        out_type=jax.ShapeDtypeStruct((_NC, _NP, F), jnp.float32),
        mesh=_sc_mesh(),
        compiler_params=pltpu.CompilerParams(use_tc_tiling_on_sc=False),
        scratch_types=[
            pltpu.VMEM((_NCHUNK, _C), jnp.int32),
            pltpu.VMEM((_NCHUNK, _C), jnp.int32),
            [pltpu.VMEM((_C, F), jnp.float32)] * nb,
            pltpu.VMEM((_ZCH, F), jnp.float32),
            pltpu.VMEM_SHARED((_NP, F), jnp.float32),
            [pltpu.SemaphoreType.DMA] * nb,
            [pltpu.SemaphoreType.DMA] * nb,
        ],
    )
    def prop(src_hbm, dst_hbm, g_hbm, acc_hbm,
             src_v, dst_v, rows, stage_v, acc_sh, gsems, ssems):
        cid = lax.axis_index("c")
        sid = lax.axis_index("s")
        wid = cid * _NS + sid

        _zero_fill(stage_v, _ZCH, F)
        base = sid * _ZCH * _NZ
        for z in range(_NZ):
            pltpu.sync_copy(stage_v, acc_sh.at[pl.ds(base + z * _ZCH, _ZCH)])
        pltpu.sync_copy(src_hbm.at[wid], src_v)
        pltpu.sync_copy(dst_hbm.at[wid], dst_v)
        plsc.subcore_barrier()

        _edge_loop(g_hbm, acc_sh, src_v, dst_v, rows, gsems, ssems, _NCHUNK,
                   nb)
        plsc.subcore_barrier()

        for z in range(_NZ):
            pltpu.sync_copy(acc_sh.at[pl.ds(base + z * _ZCH, _ZCH)], stage_v)
            pltpu.sync_copy(stage_v, acc_hbm.at[cid, pl.ds(base + z * _ZCH, _ZCH)])

    return prop


_R = 2560


def _tc_first(x, W1, deg2):
    def body(x_ref, w_ref, deg_ref, g0_ref, g1_ref, dinv_ref):
        deg = deg_ref[0] + deg_ref[1] + 1.0
        dinv = lax.rsqrt(deg)
        h = jnp.dot(x_ref[...], w_ref[...], preferred_element_type=jnp.float32)
        g = h * dinv[:, 0:1]
        g0_ref[...] = g[:, :64]
        g1_ref[...] = g[:, 64:]
        dinv_ref[...] = dinv

    return pl.pallas_call(
        body,
        grid=(_NP // _R,),
        in_specs=[
            pl.BlockSpec((_R, 128), lambda i: (i, 0)),
            pl.BlockSpec((128, 128), lambda i: (0, 0)),
            pl.BlockSpec((_NC, _R, 16), lambda i: (0, i, 0)),
        ],
        out_specs=[
            pl.BlockSpec((_R, 64), lambda i: (i, 0)),
            pl.BlockSpec((_R, 64), lambda i: (i, 0)),
            pl.BlockSpec((_R, 16), lambda i: (i, 0)),
        ],
        out_shape=[
            jax.ShapeDtypeStruct((_NP, 64), jnp.float32),
            jax.ShapeDtypeStruct((_NP, 64), jnp.float32),
            jax.ShapeDtypeStruct((_NP, 16), jnp.float32),
        ],
    )(x, W1, deg2)


def _tc_mid(acc0, acc1, g0, g1, dinv, b, W, act, F_in, F_out, split_out):
    Fh = F_in // 2
    Fo = F_out // 2 if split_out else F_out

    def body(a0_ref, a1_ref, g0_ref, g1_ref, dinv_ref, b_ref, w_ref, *o_refs):
        dv = dinv_ref[:, 0:1]
        bb = b_ref[...]
        h0 = act((a0_ref[...] + g0_ref[...]) * dv + bb[:, :Fh])
        h1 = act((a1_ref[...] + g1_ref[...]) * dv + bb[:, Fh:])
        h = jnp.concatenate([h0, h1], axis=1)
        o = jnp.dot(h, w_ref[...], preferred_element_type=jnp.float32) * dv
        if split_out:
            o_refs[0][...] = o[:, :Fo]
            o_refs[1][...] = o[:, Fo:]
        else:
            o_refs[0][...] = o

    half = lambda: pl.BlockSpec((_R, Fh), lambda i: (i, 0))
    out_spec = pl.BlockSpec((_R, Fo), lambda i: (i, 0))
    out_shape = jax.ShapeDtypeStruct((_NP, Fo), jnp.float32)
    return pl.pallas_call(
        body,
        grid=(_NP // _R,),
        in_specs=[
            half(), half(), half(), half(),
            pl.BlockSpec((_R, 16), lambda i: (i, 0)),
            pl.BlockSpec((1, F_in), lambda i: (0, 0)),
            pl.BlockSpec((F_in, F_out), lambda i: (0, 0)),
        ],
        out_specs=[out_spec, out_spec] if split_out else [out_spec],
        out_shape=[out_shape, out_shape] if split_out else [out_shape],
    )(acc0, acc1, g0, g1, dinv, b, W)


def _tc_final(acc, g, dinv, b, F):
    def body(acc_ref, g_ref, dinv_ref, b_ref, o_ref):
        o_ref[...] = ((acc_ref[0] + acc_ref[1] + g_ref[...]) * dinv_ref[:, 0:1]
                      + b_ref[...])

    return pl.pallas_call(
        body,
        grid=(_NP // _R,),
        in_specs=[
            pl.BlockSpec((_NC, _R, F), lambda i: (0, i, 0)),
            pl.BlockSpec((_R, F), lambda i: (i, 0)),
            pl.BlockSpec((_R, 16), lambda i: (i, 0)),
            pl.BlockSpec((1, F), lambda i: (0, 0)),
        ],
        out_specs=pl.BlockSpec((_R, F), lambda i: (i, 0)),
        out_shape=jax.ShapeDtypeStruct((_NP, F), jnp.float32),
    )(acc, g, dinv, b)


def kernel(x, edge_index, W1, b1, W2, b2, W3, b3, W4, b4):
    f32 = jnp.float32
    pad_idx = jnp.full((_EP - _E,), _N, jnp.int32)
    src_p = jnp.concatenate([edge_index[0], pad_idx])
    dst_p = jnp.concatenate([edge_index[1], pad_idx])
    src32 = src_p.reshape(_NW, _NCHUNK, _C)
    dst32 = dst_p.reshape(_NW, _NCHUNK, _C)
    src16 = src_p.reshape(_NS, _NCH2, _C)
    dst16 = dst_p.reshape(_NS, _NCH2, _C)

    xp = jnp.zeros((_NP, 128), f32).at[:_N].set(x)
    W2p = jnp.zeros((128, 128), f32).at[:, :100].set(W2)
    b2p = jnp.zeros((1, 128), f32).at[0, :100].set(b2)
    W3p = jnp.zeros((128, 32), f32).at[:100, :27].set(W3)
    b3p = jnp.zeros((1, 32), f32).at[0, :27].set(b3)
    W4p = jnp.zeros((32, 16), f32).at[:27, :10].set(W4)
    b4p = jnp.zeros((1, 16), f32).at[0, :10].set(b4)
    b1r = b1.reshape(1, 128)

    deg2 = _make_deg()(dst32)
    g1a, g1b, dinv = _tc_first(xp, W1, deg2)
    a1a, a1b = _make_prop_split(64)(src16, dst16, g1a, g1b)
    g2a, g2b = _tc_mid(a1a, a1b, g1a, g1b, dinv, b1r, W2p, jnp.tanh,
                       128, 128, split_out=True)
    a2a, a2b = _make_prop_split(64)(src16, dst16, g2a, g2b)
    g3a, g3b = _tc_mid(a2a, a2b, g2a, g2b, dinv, b2p, W3p, jax.nn.relu,
                       128, 32, split_out=True)
    a3a, a3b = _make_prop_split(16, nb=12)(src16, dst16, g3a, g3b)
    (g4,) = _tc_mid(a3a, a3b, g3a, g3b, dinv, b3p, W4p, jax.nn.relu,
                    32, 16, split_out=False)
    acc4 = _make_prop_full(16, nb=12)(src32, dst32, g4)
    out = _tc_final(acc4, g4, dinv, b4p, 16)
    return out[:_N, :10]


# speedup vs baseline: 25.2580x; 1.0415x over previous
import functools

import jax
import jax.numpy as jnp
from jax import lax
from jax.experimental import pallas as pl
from jax.experimental.pallas import tpu as pltpu
from jax.experimental.pallas import tpu_sc as plsc

_N = 10000
_NP = 10240
_E = 320000
_NC = 2
_NS = 16
_NW = _NC * _NS
_C = 120
_NCHUNK = 84
_EP = _NW * _NCHUNK * _C
_NCH2 = 2 * _NCHUNK
_NB = 6
_NBF = 4
_PASS = _NCHUNK
_ZCH = 128
_NZ = _NP // _NS // _ZCH


@functools.lru_cache(maxsize=None)
def _sc_mesh():
    return plsc.VectorSubcoreMesh(
        core_axis_name="c", subcore_axis_name="s", num_cores=_NC, num_subcores=_NS
    )


def _zero_fill(buf, rows, lanes):
    zero = jnp.zeros((16,), jnp.float32)

    def body(i, _):
        for k in range(lanes // 16):
            buf[i, pl.ds(k * 16, 16)] = zero
        return 0

    lax.fori_loop(0, rows, body, 0)


def _edge_loop(g_hbm, acc_sh, src_v, dst_v, rows, gsems, ssems, nchunk, nb):
    for u in range(nb):
        pltpu.async_copy(g_hbm.at[src_v.at[u]], rows[u], gsems[u])

    def body(k, _):
        base = k * nb
        scats = []
        for u in range(nb):
            jc = base + u
            pltpu.make_async_copy(g_hbm.at[src_v.at[jc]], rows[u], gsems[u]).wait()
            scats.append(
                pltpu.async_copy(rows[u], acc_sh.at[dst_v.at[jc]], ssems[u],
                                 add=True))
        for u in range(nb):
            jn = base + nb + u

            @pl.when(jn < nchunk)
            def _(u=u, jn=jn):
                scats[u].wait()
                pltpu.async_copy(g_hbm.at[src_v.at[jn]], rows[u], gsems[u])

        return 0

    lax.fori_loop(0, nchunk // nb, body, 0)
    for u in range(nb):
        pltpu.make_async_copy(
            rows[u], acc_sh.at[dst_v.at[nchunk - nb + u]], ssems[u]).wait()


@functools.lru_cache(maxsize=None)
def _make_deg():
    @functools.partial(
        pl.kernel,
        out_type=jax.ShapeDtypeStruct((_NC, _NP, 16), jnp.float32),
        mesh=_sc_mesh(),
        compiler_params=pltpu.CompilerParams(use_tc_tiling_on_sc=False),
        scratch_types=[
            pltpu.VMEM((_NCHUNK, _C), jnp.int32),
            pltpu.VMEM((_C, 16), jnp.float32),
            pltpu.VMEM((_ZCH, 16), jnp.float32),
            pltpu.VMEM_SHARED((_NP, 16), jnp.float32),
            [pltpu.SemaphoreType.DMA] * _NBF,
        ],
    )
    def _deg_kernel(dst_hbm, deg_hbm, dst_v, ones_v, stage_v, acc_sh, ssems):
        cid = lax.axis_index("c")
        sid = lax.axis_index("s")
        wid = cid * _NS + sid

        one = jnp.ones((16,), jnp.float32)

        def fill(i, _):
            ones_v[i, :] = one
            return 0

        lax.fori_loop(0, _C, fill, 0)
        _zero_fill(stage_v, _ZCH, 16)

        base = sid * _ZCH * _NZ
        for z in range(_NZ):
            pltpu.sync_copy(stage_v, acc_sh.at[pl.ds(base + z * _ZCH, _ZCH)])
        pltpu.sync_copy(dst_hbm.at[wid], dst_v)
        plsc.subcore_barrier()

        def body(k, _):
            base = k * _NBF
            for u in range(_NBF):
                jc = base + u

                @pl.when(jc >= _NBF)
                def _(u=u, jc=jc):
                    pltpu.make_async_copy(
                        ones_v, acc_sh.at[dst_v.at[jc - _NBF]], ssems[u]).wait()

                pltpu.async_copy(ones_v, acc_sh.at[dst_v.at[jc]], ssems[u],
                                 add=True)
            return 0

        lax.fori_loop(0, _NCHUNK // _NBF, body, 0)
        for u in range(_NBF):
            pltpu.make_async_copy(
                ones_v, acc_sh.at[dst_v.at[_NCHUNK - _NBF + u]], ssems[u]).wait()
        plsc.subcore_barrier()

        for z in range(_NZ):
            pltpu.sync_copy(acc_sh.at[pl.ds(base + z * _ZCH, _ZCH)], stage_v)
            pltpu.sync_copy(stage_v, deg_hbm.at[cid, pl.ds(base + z * _ZCH, _ZCH)])

    return _deg_kernel


@functools.lru_cache(maxsize=None)
def _make_prop_split(Fh, nb=_NB, pchunks=_PASS):
    @functools.partial(
        pl.kernel,
        out_type=[
            jax.ShapeDtypeStruct((_NP, Fh), jnp.float32),
            jax.ShapeDtypeStruct((_NP, Fh), jnp.float32),
        ],
        mesh=_sc_mesh(),
        compiler_params=pltpu.CompilerParams(use_tc_tiling_on_sc=False),
        scratch_types=[
            pltpu.VMEM((pchunks, _C), jnp.int32),
            pltpu.VMEM((pchunks, _C), jnp.int32),
            [pltpu.VMEM((_C, Fh), jnp.float32)] * nb,
            pltpu.VMEM((_ZCH, Fh), jnp.float32),
            pltpu.VMEM_SHARED((_NP, Fh), jnp.float32),
            [pltpu.SemaphoreType.DMA] * nb,
            [pltpu.SemaphoreType.DMA] * nb,
        ],
    )
    def prop(src_hbm, dst_hbm, g0_hbm, g1_hbm, out0_hbm, out1_hbm,
             src_v, dst_v, rows, stage_v, acc_sh, gsems, ssems):
        cid = lax.axis_index("c")
        sid = lax.axis_index("s")

        _zero_fill(stage_v, _ZCH, Fh)
        base = sid * _ZCH * _NZ
        for z in range(_NZ):
            pltpu.sync_copy(stage_v, acc_sh.at[pl.ds(base + z * _ZCH, _ZCH)])
        plsc.subcore_barrier()

        def run(g_hbm, out_hbm):
            for p in range(_NCH2 // pchunks):
                pltpu.sync_copy(src_hbm.at[sid, pl.ds(p * pchunks, pchunks)], src_v)
                pltpu.sync_copy(dst_hbm.at[sid, pl.ds(p * pchunks, pchunks)], dst_v)
                _edge_loop(g_hbm, acc_sh, src_v, dst_v, rows, gsems, ssems,
                           pchunks, nb)
            plsc.subcore_barrier()
            for z in range(_NZ):
                pltpu.sync_copy(acc_sh.at[pl.ds(base + z * _ZCH, _ZCH)], stage_v)
                pltpu.sync_copy(stage_v, out_hbm.at[pl.ds(base + z * _ZCH, _ZCH)])

        pl.when(cid == 0)(lambda: run(g0_hbm, out0_hbm))
        pl.when(cid == 1)(lambda: run(g1_hbm, out1_hbm))

    return prop


@functools.lru_cache(maxsize=None)
def _make_prop_full(F, nb=_NBF):
    @functools.partial(
        pl.kernel,
        out_type=jax.ShapeDtypeStruct((_NC, _NP, F), jnp.float32),
        mesh=_sc_mesh(),
        compiler_params=pltpu.CompilerParams(use_tc_tiling_on_sc=False),
        scratch_types=[
            pltpu.VMEM((_NCHUNK, _C), jnp.int32),
            pltpu.VMEM((_NCHUNK, _C), jnp.int32),
            [pltpu.VMEM((_C, F), jnp.float32)] * nb,
            pltpu.VMEM((_ZCH, F), jnp.float32),
            pltpu.VMEM_SHARED((_NP, F), jnp.float32),
            [pltpu.SemaphoreType.DMA] * nb,
            [pltpu.SemaphoreType.DMA] * nb,
        ],
    )
    def prop(src_hbm, dst_hbm, g_hbm, acc_hbm,
             src_v, dst_v, rows, stage_v, acc_sh, gsems, ssems):
        cid = lax.axis_index("c")
        sid = lax.axis_index("s")
        wid = cid * _NS + sid

        _zero_fill(stage_v, _ZCH, F)
        base = sid * _ZCH * _NZ
        for z in range(_NZ):
            pltpu.sync_copy(stage_v, acc_sh.at[pl.ds(base + z * _ZCH, _ZCH)])
        pltpu.sync_copy(src_hbm.at[wid], src_v)
        pltpu.sync_copy(dst_hbm.at[wid], dst_v)
        plsc.subcore_barrier()

        _edge_loop(g_hbm, acc_sh, src_v, dst_v, rows, gsems, ssems, _NCHUNK,
                   nb)
        plsc.subcore_barrier()

        for z in range(_NZ):
            pltpu.sync_copy(acc_sh.at[pl.ds(base + z * _ZCH, _ZCH)], stage_v)
            pltpu.sync_copy(stage_v, acc_hbm.at[cid, pl.ds(base + z * _ZCH, _ZCH)])

    return prop


_R = 2000


def _tc_first(x, W1, deg2):
    def body(x_ref, w_ref, deg_ref, g0_ref, g1_ref, dinv_ref):
        deg = deg_ref[0] + deg_ref[1] + 1.0
        dinv = lax.rsqrt(deg)
        h = jnp.dot(x_ref[...], w_ref[...], preferred_element_type=jnp.float32)
        g = h * dinv[:, 0:1]
        g0_ref[...] = g[:, :64]
        g1_ref[...] = g[:, 64:]
        dinv_ref[...] = dinv

    return pl.pallas_call(
        body,
        grid=(_N // _R,),
        in_specs=[
            pl.BlockSpec((_R, 128), lambda i: (i, 0)),
            pl.BlockSpec((128, 128), lambda i: (0, 0)),
            pl.BlockSpec((_NC, _R, 16), lambda i: (0, i, 0)),
        ],
        out_specs=[
            pl.BlockSpec((_R, 64), lambda i: (i, 0)),
            pl.BlockSpec((_R, 64), lambda i: (i, 0)),
            pl.BlockSpec((_R, 16), lambda i: (i, 0)),
        ],
        out_shape=[
            jax.ShapeDtypeStruct((_N, 64), jnp.float32),
            jax.ShapeDtypeStruct((_N, 64), jnp.float32),
            jax.ShapeDtypeStruct((_N, 16), jnp.float32),
        ],
    )(x, W1, deg2)


def _tc_mid(acc0, acc1, g0, g1, dinv, b, W, act, F_in, F_out, split_out):
    Fh = F_in // 2
    Fo = F_out // 2 if split_out else F_out

    def body(a0_ref, a1_ref, g0_ref, g1_ref, dinv_ref, b_ref, w_ref, *o_refs):
        dv = dinv_ref[:, 0:1]
        bb = b_ref[...]
        h0 = act((a0_ref[...] + g0_ref[...]) * dv + bb[:, :Fh])
        h1 = act((a1_ref[...] + g1_ref[...]) * dv + bb[:, Fh:])
        h = jnp.concatenate([h0, h1], axis=1)
        o = jnp.dot(h, w_ref[...], preferred_element_type=jnp.float32) * dv
        if split_out:
            o_refs[0][...] = o[:, :Fo]
            o_refs[1][...] = o[:, Fo:]
        else:
            o_refs[0][...] = o

    half = lambda: pl.BlockSpec((_R, Fh), lambda i: (i, 0))
    out_spec = pl.BlockSpec((_R, Fo), lambda i: (i, 0))
    out_shape = jax.ShapeDtypeStruct((_N, Fo), jnp.float32)
    return pl.pallas_call(
        body,
        grid=(_N // _R,),
        in_specs=[
            half(), half(), half(), half(),
            pl.BlockSpec((_R, 16), lambda i: (i, 0)),
            pl.BlockSpec((1, F_in), lambda i: (0, 0)),
            pl.BlockSpec((F_in, F_out), lambda i: (0, 0)),
        ],
        out_specs=[out_spec, out_spec] if split_out else [out_spec],
        out_shape=[out_shape, out_shape] if split_out else [out_shape],
    )(acc0, acc1, g0, g1, dinv, b, W)


def _tc_final(acc, g, dinv, b, F):
    def body(acc_ref, g_ref, dinv_ref, b_ref, o_ref):
        o_ref[...] = ((acc_ref[0] + acc_ref[1] + g_ref[...]) * dinv_ref[:, 0:1]
                      + b_ref[...])

    return pl.pallas_call(
        body,
        grid=(_N // _R,),
        in_specs=[
            pl.BlockSpec((_NC, _R, F), lambda i: (0, i, 0)),
            pl.BlockSpec((_R, F), lambda i: (i, 0)),
            pl.BlockSpec((_R, 16), lambda i: (i, 0)),
            pl.BlockSpec((1, F), lambda i: (0, 0)),
        ],
        out_specs=pl.BlockSpec((_R, F), lambda i: (i, 0)),
        out_shape=jax.ShapeDtypeStruct((_N, F), jnp.float32),
    )(acc, g, dinv, b)


def kernel(x, edge_index, W1, b1, W2, b2, W3, b3, W4, b4):
    f32 = jnp.float32
    src_p = jnp.concatenate([edge_index[0], jnp.zeros((_EP - _E,), jnp.int32)])
    dst_p = jnp.concatenate([edge_index[1], jnp.full((_EP - _E,), _N, jnp.int32)])
    src32 = src_p.reshape(_NW, _NCHUNK, _C)
    dst32 = dst_p.reshape(_NW, _NCHUNK, _C)
    src16 = src_p.reshape(_NS, _NCH2, _C)
    dst16 = dst_p.reshape(_NS, _NCH2, _C)

    W2p = jnp.zeros((128, 128), f32).at[:, :100].set(W2)
    b2p = jnp.zeros((1, 128), f32).at[0, :100].set(b2)
    W3p = jnp.zeros((128, 32), f32).at[:100, :27].set(W3)
    b3p = jnp.zeros((1, 32), f32).at[0, :27].set(b3)
    W4p = jnp.zeros((32, 16), f32).at[:27, :10].set(W4)
    b4p = jnp.zeros((1, 16), f32).at[0, :10].set(b4)
    b1r = b1.reshape(1, 128)

    deg2 = _make_deg()(dst32)
    g1a, g1b, dinv = _tc_first(x, W1, deg2)
    a1a, a1b = _make_prop_split(64, nb=8, pchunks=56)(src16, dst16, g1a, g1b)
    g2a, g2b = _tc_mid(a1a, a1b, g1a, g1b, dinv, b1r, W2p, jnp.tanh,
                       128, 128, split_out=True)
    a2a, a2b = _make_prop_split(64, nb=8, pchunks=56)(src16, dst16, g2a, g2b)
    g3a, g3b = _tc_mid(a2a, a2b, g2a, g2b, dinv, b2p, W3p, jax.nn.relu,
                       128, 32, split_out=True)
    a3a, a3b = _make_prop_split(16, nb=12)(src16, dst16, g3a, g3b)
    (g4,) = _tc_mid(a3a, a3b, g3a, g3b, dinv, b3p, W4p, jax.nn.relu,
                    32, 16, split_out=False)
    acc4 = _make_prop_full(16, nb=12)(src32, dst32, g4)
    out = _tc_final(acc4, g4, dinv, b4p, 16)
    return out[:, :10]
